# baseline (speedup 1.0000x reference)
import jax
import jax.numpy as jnp
from jax import lax
from jax.experimental import pallas as pl
from jax.experimental.pallas import tpu as pltpu
from jax.experimental.pallas import tpu_sc as plsc

V = 100000
D = 64
B = 4096
K = 50
NULL = V - 1

NC = 2
NS = 16
NW = NC * NS
BPW = B // NW
CB = 8
NCHUNK = BPW // CB
CIDX = CB * K
L = 16


def _sc_body(emb, bias, logp, rows, cols, pred_out, sums_out,
             colidx, rowidx, embcol, erow, nullrow, biasrow, logprow,
             biascol, logpcol, outbuf, sumsbuf, sem_a, sem_b):
    wid = lax.axis_index("s") * NC + lax.axis_index("c")
    base = wid * BPW
    lanes = lax.iota(jnp.int32, 16)

    pltpu.sync_copy(rows.at[pl.ds(base, BPW)], rowidx)
    pltpu.sync_copy(cols.at[pl.ds(base * K, BPW * K)], colidx)
    pltpu.sync_copy(emb.at[pl.ds(NULL, 1)], nullrow)
    h1 = pltpu.async_copy(emb.at[rowidx], erow, sem_a)
    h2 = pltpu.async_copy(bias.at[rowidx], biasrow, sem_a)
    h3 = pltpu.async_copy(logp.at[rowidx], logprow, sem_a)
    h1.wait()
    h2.wait()
    h3.wait()

    def _unpack_row(ref, r, off):
        out = []
        for j in range(D // 32):
            half = ref[r, pl.ds(off + j * 32, 32)]
            a, b = plsc.unpack(half, format=plsc.PackFormat.INTERLEAVED)
            out.append(a)
            out.append(b)
        return out

    nvec = _unpack_row(nullrow, 0, 0)

    rs = logprow[pl.ds(0, L)]
    for i in range(1, BPW // L):
        rs = rs + logprow[pl.ds(i * L, L)]
    sumsbuf[pl.ds(0, L)] = rs
    sumsbuf[pl.ds(L, L)] = jnp.zeros((L,), jnp.float32)

    def _fire(i, soff, sem):
        idxsl = colidx.at[pl.ds(i * CIDX, CIDX)]
        pltpu.async_copy(emb.at[idxsl], embcol.at[pl.ds(soff, CIDX)], sem)
        pltpu.async_copy(bias.at[idxsl], biascol.at[pl.ds(soff, CIDX)], sem)
        pltpu.async_copy(logp.at[idxsl], logpcol.at[pl.ds(soff, CIDX)], sem)

    def _drain(soff, sem):
        pltpu.make_async_copy(emb.at[pl.ds(0, CIDX)],
                              embcol.at[pl.ds(soff, CIDX)], sem).wait()
        pltpu.make_async_copy(bias.at[pl.ds(0, CIDX)],
                              biascol.at[pl.ds(soff, CIDX)], sem).wait()
        pltpu.make_async_copy(logp.at[pl.ds(0, CIDX)],
                              logpcol.at[pl.ds(soff, CIDX)], sem).wait()

    def _compute(c, soff):
        @pl.loop(0, CIDX // L)
        def _ls(i):
            plsc.addupdate(sumsbuf.at[pl.ds(L, L)],
                           logpcol[pl.ds(soff + i * L, L)])

        @pl.loop(0, CB)
        def _b(bi):
            bl = c * CB + bi
            blv = jnp.full((L,), bl, jnp.int32)
            evec = _unpack_row(erow, bl, 0)
            avec = [evec[j] - nvec[j] for j in range(D // L)]
            br = plsc.load_gather(biasrow, [blv])
            sbp = None
            for j in range(D // L):
                t = avec[j] * nvec[j]
                sbp = t if sbp is None else sbp + t
            sb = jnp.sum(sbp)
            cbase = jnp.full((L,), soff + bi * K, jnp.int32)
            for g in range(4):
                acc = jnp.zeros((L,), jnp.float32)
                for p in range(16):
                    k = g * 16 + p
                    if k >= K:
                        break
                    crow = soff + bi * K + k
                    cvec = _unpack_row(embcol, crow, 0)
                    prod = None
                    for j in range(D // L):
                        t = avec[j] * cvec[j]
                        prod = t if prod is None else prod + t
                    tot = jnp.sum(prod)
                    acc = jnp.where(lanes == p, tot, acc)
                kvec = lanes + g * 16
                kmask = kvec < K
                kcl = jnp.minimum(kvec, K - 1)
                bc = plsc.load_gather(biascol, [cbase + kcl])
                plsc.store_scatter(outbuf, [blv * K + kvec],
                                   acc - sb + br + bc, mask=kmask)

    _fire(0, 0, sem_a)

    @pl.loop(0, NCHUNK, step=2)
    def _c2(c):
        _fire(c + 1, CIDX, sem_b)
        _drain(0, sem_a)
        _compute(c, 0)

        @pl.when(c + 2 < NCHUNK)
        def _():
            _fire(c + 2, 0, sem_a)

        _drain(CIDX, sem_b)
        _compute(c + 1, CIDX)

    pltpu.sync_copy(outbuf, pred_out.at[pl.ds(base * K, BPW * K)])
    pltpu.sync_copy(sumsbuf, sums_out.at[pl.ds(wid * 32, 32)])


def _tc_body(logp_ref, sums_ref, out_ref):
    s_all = jnp.sum(logp_ref[...])
    lane = lax.broadcasted_iota(jnp.int32, (8, 128), 1)
    s = sums_ref[...]
    rowtot = jnp.sum(jnp.where((lane % 32) < 16, s, 0.0))
    coltot = jnp.sum(jnp.where((lane % 32) >= 16, s, 0.0))
    null_lp = logp_ref[NULL]
    val = 0.5 * (rowtot / B + coltot / (B * K) + null_lp + s_all / V)
    out_ref[...] = jnp.broadcast_to(val, (1, 1))


def kernel(emb, biases, logp, row_indices, col_matrix):
    bias_flat = biases.reshape(V)
    cols_flat = jnp.maximum(col_matrix.reshape(B * K), 0)
    rows = row_indices
    emb = emb.astype(jnp.bfloat16)

    mesh = plsc.VectorSubcoreMesh(core_axis_name="c", subcore_axis_name="s",
                                  num_cores=NC, num_subcores=NS)
    predflat, sums = pl.kernel(
        _sc_body,
        out_type=(jax.ShapeDtypeStruct((B * K,), jnp.float32),
                  jax.ShapeDtypeStruct((NW * 32,), jnp.float32)),
        mesh=mesh,
        compiler_params=pltpu.CompilerParams(needs_layout_passes=False,
                                             use_tc_tiling_on_sc=False),
        scratch_types=[
            pltpu.VMEM((BPW * K,), jnp.int32),
            pltpu.VMEM((BPW,), jnp.int32),
            pltpu.VMEM((2 * CIDX, D), jnp.bfloat16),
            pltpu.VMEM((BPW, D), jnp.bfloat16),
            pltpu.VMEM((1, D), jnp.bfloat16),
            pltpu.VMEM((BPW,), jnp.float32),
            pltpu.VMEM((BPW,), jnp.float32),
            pltpu.VMEM((2 * CIDX,), jnp.float32),
            pltpu.VMEM((2 * CIDX,), jnp.float32),
            pltpu.VMEM((BPW * K,), jnp.float32),
            pltpu.VMEM((32,), jnp.float32),
            pltpu.SemaphoreType.DMA,
            pltpu.SemaphoreType.DMA,
        ],
    )(emb, bias_flat, logp, rows, cols_flat)

    sums2d = sums.reshape(8, 128)
    mscal = pl.pallas_call(
        _tc_body,
        out_shape=jax.ShapeDtypeStruct((1, 1), jnp.float32),
    )(logp, sums2d)
    return (predflat.reshape(B, K), mscal[0, 0])


# speedup vs baseline: 1.0008x; 1.0008x over previous
import jax
import jax.numpy as jnp
from jax import lax
from jax.experimental import pallas as pl
from jax.experimental.pallas import tpu as pltpu
from jax.experimental.pallas import tpu_sc as plsc

V = 100000
D = 64
B = 4096
K = 50
NULL = V - 1

NC = 2
NS = 16
NW = NC * NS
BPW = B // NW
CB = 8
NCHUNK = BPW // CB
CIDX = CB * K
L = 16


def _sc_body(emb, bias, logp, rows, cols, pred_out, sums_out,
             colidx, rowidx, embcol, erow, nullrow, biasrow, logprow,
             biascol, logpcol, outbuf, sumsbuf, sem_a, sem_b):
    wid = lax.axis_index("s") * NC + lax.axis_index("c")
    base = wid * BPW
    lanes = lax.iota(jnp.int32, 16)

    pltpu.sync_copy(rows.at[pl.ds(base, BPW)], rowidx)
    pltpu.sync_copy(cols.at[pl.ds(base * K, BPW * K)], colidx)
    pltpu.sync_copy(emb.at[pl.ds(NULL, 1)], nullrow)
    h1 = pltpu.async_copy(emb.at[rowidx], erow, sem_a)
    h2 = pltpu.async_copy(bias.at[rowidx], biasrow, sem_a)
    h3 = pltpu.async_copy(logp.at[rowidx], logprow, sem_a)
    h1.wait()
    h2.wait()
    h3.wait()

    def _unpack_row(ref, r, off):
        out = []
        for j in range(D // 32):
            half = ref[r, pl.ds(off + j * 32, 32)]
            a, b = plsc.unpack(half, format=plsc.PackFormat.INTERLEAVED)
            out.append(a)
            out.append(b)
        return out

    nvec = _unpack_row(nullrow, 0, 0)

    rs = logprow[pl.ds(0, L)]
    for i in range(1, BPW // L):
        rs = rs + logprow[pl.ds(i * L, L)]
    sumsbuf[pl.ds(0, L)] = rs
    sumsbuf[pl.ds(L, L)] = jnp.zeros((L,), jnp.float32)

    def _fire(i, soff, sem):
        idxsl = colidx.at[pl.ds(i * CIDX, CIDX)]
        pltpu.async_copy(emb.at[idxsl], embcol.at[pl.ds(soff, CIDX)], sem)
        pltpu.async_copy(bias.at[idxsl], biascol.at[pl.ds(soff, CIDX)], sem)
        pltpu.async_copy(logp.at[idxsl], logpcol.at[pl.ds(soff, CIDX)], sem)

    def _drain(soff, sem):
        pltpu.make_async_copy(emb.at[pl.ds(0, CIDX)],
                              embcol.at[pl.ds(soff, CIDX)], sem).wait()
        pltpu.make_async_copy(bias.at[pl.ds(0, CIDX)],
                              biascol.at[pl.ds(soff, CIDX)], sem).wait()
        pltpu.make_async_copy(logp.at[pl.ds(0, CIDX)],
                              logpcol.at[pl.ds(soff, CIDX)], sem).wait()

    def _compute(c, soff):
        @pl.loop(0, CIDX // L)
        def _ls(i):
            plsc.addupdate(sumsbuf.at[pl.ds(L, L)],
                           logpcol[pl.ds(soff + i * L, L)])

        @pl.loop(0, CB)
        def _b(bi):
            bl = c * CB + bi
            blv = jnp.full((L,), bl, jnp.int32)
            evec = _unpack_row(erow, bl, 0)
            avec = [evec[j] - nvec[j] for j in range(D // L)]
            br = plsc.load_gather(biasrow, [blv])
            sbp = None
            for j in range(D // L):
                t = avec[j] * nvec[j]
                sbp = t if sbp is None else sbp + t
            sb = jnp.sum(sbp)
            cbase = jnp.full((L,), soff + bi * K, jnp.int32)
            for g in range(4):
                acc = jnp.zeros((L,), jnp.float32)
                for p in range(16):
                    k = g * 16 + p
                    if k >= K:
                        break
                    crow = soff + bi * K + k
                    cvec = _unpack_row(embcol, crow, 0)
                    prod = None
                    for j in range(D // L):
                        t = avec[j] * cvec[j]
                        prod = t if prod is None else prod + t
                    tot = jnp.sum(prod)
                    acc = jnp.where(lanes == p, tot, acc)
                kvec = lanes + g * 16
                kmask = kvec < K
                kcl = jnp.minimum(kvec, K - 1)
                bc = plsc.load_gather(biascol, [cbase + kcl])
                plsc.store_scatter(outbuf, [blv * K + kvec],
                                   acc - sb + br + bc, mask=kmask)

    _fire(0, 0, sem_a)

    @pl.loop(0, NCHUNK, step=2)
    def _c2(c):
        _fire(c + 1, CIDX, sem_b)
        _drain(0, sem_a)
        _compute(c, 0)

        @pl.when(c + 2 < NCHUNK)
        def _():
            _fire(c + 2, 0, sem_a)

        _drain(CIDX, sem_b)
        _compute(c + 1, CIDX)

    pltpu.sync_copy(outbuf, pred_out.at[pl.ds(base * K, BPW * K)])
    pltpu.sync_copy(sumsbuf, sums_out.at[pl.ds(wid * 32, 32)])


def _tc_body(logp_ref, sums_ref, out_ref):
    s_all = jnp.sum(logp_ref[...])
    lane = lax.broadcasted_iota(jnp.int32, (8, 128), 1)
    s = sums_ref[...]
    rowtot = jnp.sum(jnp.where((lane % 32) < 16, s, 0.0))
    coltot = jnp.sum(jnp.where((lane % 32) >= 16, s, 0.0))
    null_lp = logp_ref[NULL]
    val = 0.5 * (rowtot / B + coltot / (B * K) + null_lp + s_all / V)
    out_ref[...] = jnp.broadcast_to(val, (1, 1))


def kernel(emb, biases, logp, row_indices, col_matrix):
    bias_flat = biases.reshape(V)
    cols_flat = jnp.maximum(col_matrix.reshape(B * K), 0)
    rows = row_indices
    emb = lax.optimization_barrier(emb.astype(jnp.bfloat16))

    mesh = plsc.VectorSubcoreMesh(core_axis_name="c", subcore_axis_name="s",
                                  num_cores=NC, num_subcores=NS)
    predflat, sums = pl.kernel(
        _sc_body,
        out_type=(jax.ShapeDtypeStruct((B * K,), jnp.float32),
                  jax.ShapeDtypeStruct((NW * 32,), jnp.float32)),
        mesh=mesh,
        compiler_params=pltpu.CompilerParams(needs_layout_passes=False,
                                             use_tc_tiling_on_sc=False),
        scratch_types=[
            pltpu.VMEM((BPW * K,), jnp.int32),
            pltpu.VMEM((BPW,), jnp.int32),
            pltpu.VMEM((2 * CIDX, D), jnp.bfloat16),
            pltpu.VMEM((BPW, D), jnp.bfloat16),
            pltpu.VMEM((1, D), jnp.bfloat16),
            pltpu.VMEM((BPW,), jnp.float32),
            pltpu.VMEM((BPW,), jnp.float32),
            pltpu.VMEM((2 * CIDX,), jnp.float32),
            pltpu.VMEM((2 * CIDX,), jnp.float32),
            pltpu.VMEM((BPW * K,), jnp.float32),
            pltpu.VMEM((32,), jnp.float32),
            pltpu.SemaphoreType.DMA,
            pltpu.SemaphoreType.DMA,
        ],
    )(emb, bias_flat, logp, rows, cols_flat)

    sums2d = sums.reshape(8, 128)
    mscal = pl.pallas_call(
        _tc_body,
        out_shape=jax.ShapeDtypeStruct((1, 1), jnp.float32),
    )(logp, sums2d)
    return (predflat.reshape(B, K), mscal[0, 0])


# speedup vs baseline: 1.2520x; 1.2511x over previous
import jax
import jax.numpy as jnp
from jax import lax
from jax.experimental import pallas as pl
from jax.experimental.pallas import tpu as pltpu
from jax.experimental.pallas import tpu_sc as plsc

V = 100000
D = 64
B = 4096
K = 50
NULL = V - 1

NC = 2
NS = 16
NW = NC * NS
BPW = B // NW
CB = 8
NCHUNK = BPW // CB
CIDX = CB * K
L = 16


def _sc_body(emb, bias, logp, rows, cols, rows2, cols2, pred_out, sums_out,
             colidx, colidx2, rowidx, rowidx2, embcol, erow, nullrow,
             biasrow, logprow, biascol, logpcol, outbuf, sumsbuf,
             sem_a, sem_b):
    wid = lax.axis_index("s") * NC + lax.axis_index("c")
    base = wid * BPW
    lanes = lax.iota(jnp.int32, 16)

    pltpu.sync_copy(rows.at[pl.ds(base, BPW)], rowidx)
    pltpu.sync_copy(rows2.at[pl.ds(base, BPW)], rowidx2)
    pltpu.sync_copy(cols.at[pl.ds(base * K, BPW * K)], colidx)
    pltpu.sync_copy(cols2.at[pl.ds(base * K, BPW * K)], colidx2)
    pltpu.sync_copy(emb.at[pl.ds(2 * NULL, 1)], nullrow)
    h1 = pltpu.async_copy(emb.at[rowidx2], erow, sem_a)
    h2 = pltpu.async_copy(bias.at[rowidx], biasrow, sem_a)
    h3 = pltpu.async_copy(logp.at[rowidx], logprow, sem_a)
    h1.wait()
    h2.wait()
    h3.wait()

    nvec = [nullrow[0, pl.ds(j * L, L)] for j in range(D // L)]

    rs = logprow[pl.ds(0, L)]
    for i in range(1, BPW // L):
        rs = rs + logprow[pl.ds(i * L, L)]
    sumsbuf[pl.ds(0, L)] = rs
    sumsbuf[pl.ds(L, L)] = jnp.zeros((L,), jnp.float32)

    def _fire(i, soff, sem):
        idxsl = colidx.at[pl.ds(i * CIDX, CIDX)]
        idxsl2 = colidx2.at[pl.ds(i * CIDX, CIDX)]
        pltpu.async_copy(emb.at[idxsl2], embcol.at[pl.ds(soff, CIDX)], sem)
        pltpu.async_copy(bias.at[idxsl], biascol.at[pl.ds(soff, CIDX)], sem)
        pltpu.async_copy(logp.at[idxsl], logpcol.at[pl.ds(soff, CIDX)], sem)

    def _drain(soff, sem):
        pltpu.make_async_copy(emb.at[pl.ds(0, CIDX)],
                              embcol.at[pl.ds(soff, CIDX)], sem).wait()
        pltpu.make_async_copy(bias.at[pl.ds(0, CIDX)],
                              biascol.at[pl.ds(soff, CIDX)], sem).wait()
        pltpu.make_async_copy(logp.at[pl.ds(0, CIDX)],
                              logpcol.at[pl.ds(soff, CIDX)], sem).wait()

    def _compute(c, soff):
        @pl.loop(0, CIDX // L)
        def _ls(i):
            plsc.addupdate(sumsbuf.at[pl.ds(L, L)],
                           logpcol[pl.ds(soff + i * L, L)])

        @pl.loop(0, CB)
        def _b(bi):
            bl = c * CB + bi
            blv = jnp.full((L,), bl, jnp.int32)
            avec = [erow[bl, pl.ds(j * L, L)] - nvec[j]
                    for j in range(D // L)]
            br = plsc.load_gather(biasrow, [blv])
            sbp = None
            for j in range(D // L):
                t = avec[j] * nvec[j]
                sbp = t if sbp is None else sbp + t
            sb = jnp.sum(sbp)
            cbase = jnp.full((L,), soff + bi * K, jnp.int32)
            for g in range(4):
                acc = jnp.zeros((L,), jnp.float32)
                for p in range(16):
                    k = g * 16 + p
                    if k >= K:
                        break
                    crow = soff + bi * K + k
                    prod = None
                    for j in range(D // L):
                        cj = embcol[crow, pl.ds(j * L, L)]
                        t = avec[j] * cj
                        prod = t if prod is None else prod + t
                    tot = jnp.sum(prod)
                    acc = jnp.where(lanes == p, tot, acc)
                kvec = lanes + g * 16
                kmask = kvec < K
                kcl = jnp.minimum(kvec, K - 1)
                bc = plsc.load_gather(biascol, [cbase + kcl])
                plsc.store_scatter(outbuf, [blv * K + kvec],
                                   acc - sb + br + bc, mask=kmask)

    _fire(0, 0, sem_a)

    @pl.loop(0, NCHUNK, step=2)
    def _c2(c):
        _fire(c + 1, CIDX, sem_b)
        _drain(0, sem_a)
        _compute(c, 0)

        @pl.when(c + 2 < NCHUNK)
        def _():
            _fire(c + 2, 0, sem_a)

        _drain(CIDX, sem_b)
        _compute(c + 1, CIDX)

    pltpu.sync_copy(outbuf, pred_out.at[pl.ds(base * K, BPW * K)])
    pltpu.sync_copy(sumsbuf, sums_out.at[pl.ds(wid * 32, 32)])


_RB = 2048
_RGRID = (V + _RB - 1) // _RB


def _repack_body(et_ref, out_ref):
    t = et_ref[...].T
    out_ref[...] = jnp.concatenate(
        [t, jnp.zeros((_RB, D), jnp.float32)], axis=1)


def _tc_body(logp_ref, sums_ref, out_ref):
    s_all = jnp.sum(logp_ref[...])
    lane = lax.broadcasted_iota(jnp.int32, (8, 128), 1)
    s = sums_ref[...]
    rowtot = jnp.sum(jnp.where((lane % 32) < 16, s, 0.0))
    coltot = jnp.sum(jnp.where((lane % 32) >= 16, s, 0.0))
    null_lp = logp_ref[NULL]
    val = 0.5 * (rowtot / B + coltot / (B * K) + null_lp + s_all / V)
    out_ref[...] = jnp.broadcast_to(val, (1, 1))


def kernel(emb, biases, logp, row_indices, col_matrix):
    bias_flat = biases.reshape(V)
    cols_flat = jnp.maximum(col_matrix.reshape(B * K), 0)
    rows = row_indices

    emb_wide = pl.pallas_call(
        _repack_body,
        grid=(_RGRID,),
        in_specs=[pl.BlockSpec((D, _RB), lambda g: (0, g))],
        out_specs=pl.BlockSpec((_RB, 2 * D), lambda g: (g, 0)),
        out_shape=jax.ShapeDtypeStruct((V, 2 * D), jnp.float32),
    )(emb.T)
    emb = emb_wide.reshape(2 * V, D)
    rows2 = rows + rows
    cols2_flat = cols_flat + cols_flat

    mesh = plsc.VectorSubcoreMesh(core_axis_name="c", subcore_axis_name="s",
                                  num_cores=NC, num_subcores=NS)
    predflat, sums = pl.kernel(
        _sc_body,
        out_type=(jax.ShapeDtypeStruct((B * K,), jnp.float32),
                  jax.ShapeDtypeStruct((NW * 32,), jnp.float32)),
        mesh=mesh,
        compiler_params=pltpu.CompilerParams(needs_layout_passes=False,
                                             use_tc_tiling_on_sc=False),
        scratch_types=[
            pltpu.VMEM((BPW * K,), jnp.int32),
            pltpu.VMEM((BPW * K,), jnp.int32),
            pltpu.VMEM((BPW,), jnp.int32),
            pltpu.VMEM((BPW,), jnp.int32),
            pltpu.VMEM((2 * CIDX, D), jnp.float32),
            pltpu.VMEM((BPW, D), jnp.float32),
            pltpu.VMEM((1, D), jnp.float32),
            pltpu.VMEM((BPW,), jnp.float32),
            pltpu.VMEM((BPW,), jnp.float32),
            pltpu.VMEM((2 * CIDX,), jnp.float32),
            pltpu.VMEM((2 * CIDX,), jnp.float32),
            pltpu.VMEM((BPW * K,), jnp.float32),
            pltpu.VMEM((32,), jnp.float32),
            pltpu.SemaphoreType.DMA,
            pltpu.SemaphoreType.DMA,
        ],
    )(emb, bias_flat, logp, rows, cols_flat, rows2, cols2_flat)

    sums2d = sums.reshape(8, 128)
    mscal = pl.pallas_call(
        _tc_body,
        out_shape=jax.ShapeDtypeStruct((1, 1), jnp.float32),
    )(logp, sums2d)
    return (predflat.reshape(B, K), mscal[0, 0])
